# Initial kernel scaffold; baseline (speedup 1.0000x reference)
import jax
import jax.numpy as jnp
from jax.experimental import pallas as pl


def kernel(logits, targets, metadata):
    raise NotImplementedError("write your pallas kernel here")


# speedup vs baseline: 16.9174x; 16.9174x over previous
import functools

import jax
import jax.numpy as jnp
from jax.experimental import pallas as pl
from jax.experimental.pallas import tpu as pltpu

_ALPHA = 0.5
_BETA = 0.5
_SMOOTH = 1.0
_TOP_PERCENT = 0.05
_TAU = 0.1
_BCE_WEIGHT = 0.5
_FOCAL_WEIGHT = 0.5
_EPS = 1e-12

_SELECT_BITS = 32


def _loss_kernel(k, logits_ref, targets_ref, out_ref):
    x = logits_ref[...]
    t = targets_ref[...].astype(jnp.float32)

    b = jax.lax.bitcast_convert_type(x, jnp.int32)
    s = jnp.where(b >= 0, b, b ^ jnp.int32(0x7FFFFFFF))

    def body(i, prefix):
        j = 31 - i
        cand_u = prefix | jnp.left_shift(jnp.int32(1), j)
        cand_s = cand_u ^ jnp.int32(-2147483648)
        c = jnp.sum((s >= cand_s).astype(jnp.int32))
        return jnp.where(c >= k, cand_u, prefix)

    p_u = jax.lax.fori_loop(0, _SELECT_BITS, body, jnp.int32(0))
    s_k = p_u ^ jnp.int32(-2147483648)
    b_k = jnp.where(s_k >= 0, s_k, s_k ^ jnp.int32(0x7FFFFFFF))
    x_k = jax.lax.bitcast_convert_type(b_k, jnp.float32)
    q = 1.0 / (1.0 + jnp.exp(-x_k))

    p = 1.0 / (1.0 + jnp.exp(-x))
    m = 1.0 / (1.0 + jnp.exp((q - p) / _TAU))
    p_c = jnp.clip(p, _EPS, 1.0 - _EPS)
    bce = -(t * jnp.log(p_c) + (1.0 - t) * jnp.log(1.0 - p_c))
    one_minus_pt = jnp.where(t == 1.0, 1.0 - p, p)
    focal = one_minus_pt * one_minus_pt * bce

    sum_t = jnp.sum(t)
    sum_m = jnp.sum(m)
    sum_mt = jnp.sum(m * t)
    sum_bce = jnp.sum(bce)
    sum_focal = jnp.sum(focal)

    n = jnp.float32(x.size)
    tp = sum_mt
    fp = sum_m - sum_mt
    fn = sum_t - sum_mt
    tversky = (tp + _SMOOTH) / (tp + _ALPHA * fp + _BETA * fn + _SMOOTH)
    loss = (1.0 - tversky) + _BCE_WEIGHT * sum_bce / n + _FOCAL_WEIGHT * sum_focal / n
    out_ref[0, 0] = loss


def kernel(logits, targets, metadata=0):
    n = logits.size
    k = max(1, int(_TOP_PERCENT * n))
    out = pl.pallas_call(
        functools.partial(_loss_kernel, k),
        out_shape=jax.ShapeDtypeStruct((1, 1), jnp.float32),
        out_specs=pl.BlockSpec(memory_space=pltpu.SMEM),
    )(logits, targets)
    return out[0, 0]


# speedup vs baseline: 23.6367x; 1.3972x over previous
import functools

import jax
import jax.numpy as jnp
from jax.experimental import pallas as pl
from jax.experimental.pallas import tpu as pltpu

_ALPHA = 0.5
_BETA = 0.5
_SMOOTH = 1.0
_TOP_PERCENT = 0.05
_TAU = 0.1
_BCE_WEIGHT = 0.5
_FOCAL_WEIGHT = 0.5
_EPS = 1e-12

_SELECT_BITS = 20


def _loss_kernel(k, logits_ref, targets_ref, out_ref):
    x = logits_ref[...]
    t = targets_ref[...].astype(jnp.float32)

    b = jax.lax.bitcast_convert_type(x, jnp.int32)
    s = jnp.where(b >= 0, b, b ^ jnp.int32(0x7FFFFFFF))

    def body(i, prefix):
        j = 31 - i
        cand_u = prefix | jnp.left_shift(jnp.int32(1), j)
        cand_s = cand_u ^ jnp.int32(-2147483648)
        c = jnp.sum((s >= cand_s).astype(jnp.int32))
        return jnp.where(c >= k, cand_u, prefix)

    p_u = jax.lax.fori_loop(0, _SELECT_BITS, body, jnp.int32(0))
    s_k = p_u ^ jnp.int32(-2147483648)
    b_k = jnp.where(s_k >= 0, s_k, s_k ^ jnp.int32(0x7FFFFFFF))
    x_k = jax.lax.bitcast_convert_type(b_k, jnp.float32)
    q = 1.0 / (1.0 + jnp.exp(-x_k))

    p = 1.0 / (1.0 + jnp.exp(-x))
    m = 1.0 / (1.0 + jnp.exp((q - p) / _TAU))
    p_c = jnp.clip(p, _EPS, 1.0 - _EPS)
    bce = -(t * jnp.log(p_c) + (1.0 - t) * jnp.log(1.0 - p_c))
    one_minus_pt = jnp.where(t == 1.0, 1.0 - p, p)
    focal = one_minus_pt * one_minus_pt * bce

    sum_t = jnp.sum(t)
    sum_m = jnp.sum(m)
    sum_mt = jnp.sum(m * t)
    sum_bce = jnp.sum(bce)
    sum_focal = jnp.sum(focal)

    n = jnp.float32(x.size)
    tp = sum_mt
    fp = sum_m - sum_mt
    fn = sum_t - sum_mt
    tversky = (tp + _SMOOTH) / (tp + _ALPHA * fp + _BETA * fn + _SMOOTH)
    loss = (1.0 - tversky) + _BCE_WEIGHT * sum_bce / n + _FOCAL_WEIGHT * sum_focal / n
    out_ref[0, 0] = loss


def kernel(logits, targets, metadata=0):
    n = logits.size
    k = max(1, int(_TOP_PERCENT * n))
    out = pl.pallas_call(
        functools.partial(_loss_kernel, k),
        out_shape=jax.ShapeDtypeStruct((1, 1), jnp.float32),
        out_specs=pl.BlockSpec(memory_space=pltpu.SMEM),
    )(logits, targets)
    return out[0, 0]
